# Initial kernel scaffold; baseline (speedup 1.0000x reference)
import jax
import jax.numpy as jnp
from jax.experimental import pallas as pl


def kernel(vox_feats, pts_coors, Wpos, W1, W2, b1, b2, vox_coors, vox_numbs):
    raise NotImplementedError("write your pallas kernel here")


# speedup vs baseline: 2.2967x; 2.2967x over previous
import functools

import jax
import jax.numpy as jnp
from jax import lax
from jax.experimental import pallas as pl
from jax.experimental.pallas import tpu as pltpu
from jax.experimental.pallas import tpu_sc as plsc

R, C = 256, 128
N = R * C
LOG2N = 15
D = 128
GRP = 64
ORD = 7

NW = 32
BPW = N // NW
ICH = 128


def _xor_partner(a, d):
    if d < C:
        bit = (lax.broadcasted_iota(jnp.int32, (R, C), 1) & d) != 0
        return jnp.where(bit, jnp.roll(a, d, axis=1), jnp.roll(a, -d, axis=1))
    m = d // C
    bit = (lax.broadcasted_iota(jnp.int32, (R, C), 0) & m) != 0
    return jnp.where(bit, jnp.roll(a, m, axis=0), jnp.roll(a, -m, axis=0))


def _bitonic(key, val, aux=None):
    i = (lax.broadcasted_iota(jnp.int32, (R, C), 0) * C
         + lax.broadcasted_iota(jnp.int32, (R, C), 1))
    for k in range(1, LOG2N + 1):
        asc = (i & (1 << k)) == 0
        for j in range(k - 1, -1, -1):
            d = 1 << j
            kp = _xor_partner(key, d)
            vp = _xor_partner(val, d)
            bit = (i & d) != 0
            g = (key > kp) | ((key == kp) & (val > vp))
            keep = (asc ^ bit) ^ g
            key = jnp.where(keep, key, kp)
            val = jnp.where(keep, val, vp)
            if aux is not None:
                aux = jnp.where(keep, aux, _xor_partner(aux, d))
    return key, val, aux


def _morton(b, x, y, z):
    code = jnp.zeros_like(x)
    for i in range(ORD):
        code = (code
                | (((x >> i) & 1) << (3 * i))
                | (((y >> i) & 1) << (3 * i + 1))
                | (((z >> i) & 1) << (3 * i + 2)))
    return code | (b << (3 * ORD))


def _sort_a_body(b_ref, x_ref, y_ref, z_ref, ind1_ref, c2c_ref):
    b = b_ref[...]
    x = x_ref[...]
    y = y_ref[...]
    z = z_ref[...]
    code1 = _morton(b, x, y, z)
    code2 = _morton(b, x, y + 1, z + 1)
    iota = (lax.broadcasted_iota(jnp.int32, (R, C), 0) * C
            + lax.broadcasted_iota(jnp.int32, (R, C), 1))
    _, ind1, c2c = _bitonic(code1, iota, code2)
    ind1_ref[...] = ind1
    c2c_ref[...] = c2c


def _sort_b_body(c2c_ref, ind1_ref, ind2_ref, ind12_ref):
    iota = (lax.broadcasted_iota(jnp.int32, (R, C), 0) * C
            + lax.broadcasted_iota(jnp.int32, (R, C), 1))
    packed = (ind1_ref[...] << 15) | iota
    _, sv, _ = _bitonic(c2c_ref[...], packed)
    ind2_ref[...] = sv >> 15
    ind12_ref[...] = sv & (N - 1)


def _sort_a(b2, x2, y2, z2, interpret=False):
    return pl.pallas_call(
        _sort_a_body,
        out_shape=(jax.ShapeDtypeStruct((R, C), jnp.int32),
                   jax.ShapeDtypeStruct((R, C), jnp.int32)),
        interpret=interpret,
    )(b2, x2, y2, z2)


def _sort_b(c2c, ind1, interpret=False):
    return pl.pallas_call(
        _sort_b_body,
        out_shape=(jax.ShapeDtypeStruct((R, C), jnp.int32),
                   jax.ShapeDtypeStruct((R, C), jnp.int32)),
        interpret=interpret,
    )(c2c, ind1)


RB = 2048


def _mlp_body(x_ref, p_ref, wpos_ref, w1_ref, w2_ref, b1_ref, b2_ref, o_ref):
    x = x_ref[...]
    p = p_ref[...]
    pg = p.reshape(RB // GRP, GRP, 16)
    cp = (pg - jnp.mean(pg, axis=1, keepdims=True)).reshape(RB, 16)
    e = (cp[:, 0:1] * wpos_ref[0:1, :]
         + cp[:, 1:2] * wpos_ref[1:2, :]
         + cp[:, 2:3] * wpos_ref[2:3, :])
    h = x * e
    h = jnp.dot(h, w1_ref[...], preferred_element_type=jnp.float32) + b1_ref[...]
    h = jax.nn.gelu(h)
    hg = h.reshape(RB // GRP, GRP, D)
    h = (hg - jnp.mean(hg, axis=1, keepdims=True)).reshape(RB, D)
    h = jnp.dot(h, w2_ref[...], preferred_element_type=jnp.float32) + b2_ref[...]
    o_ref[...] = x + h


def _mlp(x, p16, wpos, w1, w2, b1, b2, interpret=False):
    grid = (N // RB,)
    return pl.pallas_call(
        _mlp_body,
        grid=grid,
        in_specs=[
            pl.BlockSpec((RB, D), lambda i: (i, 0)),
            pl.BlockSpec((RB, 16), lambda i: (i, 0)),
            pl.BlockSpec((8, D), lambda i: (0, 0)),
            pl.BlockSpec((D, D), lambda i: (0, 0)),
            pl.BlockSpec((D, D), lambda i: (0, 0)),
            pl.BlockSpec((1, D), lambda i: (0, 0)),
            pl.BlockSpec((1, D), lambda i: (0, 0)),
        ],
        out_specs=pl.BlockSpec((RB, D), lambda i: (i, 0)),
        out_shape=jax.ShapeDtypeStruct((N, D), jnp.float32),
        interpret=interpret,
    )(x, p16, wpos, w1, w2, b1, b2)


def _sc_mesh():
    return plsc.VectorSubcoreMesh(core_axis_name="c", subcore_axis_name="s")


def _worker_id():
    return lax.axis_index("s") * 2 + lax.axis_index("c")


def _gather_rows(table_hbm, idx_v, out_hbm, rows_v, sem, base):
    for t in range(BPW // ICH):
        pltpu.async_copy(table_hbm.at[idx_v.at[t]], rows_v, sem).wait()
        pltpu.sync_copy(rows_v, out_hbm.at[pl.ds(base + t * ICH, ICH)])


def _wide_body(table, ind, out, idx_v, rows_v, sem):
    wid = _worker_id()
    pltpu.sync_copy(ind.at[pl.ds(wid * (BPW // C), BPW // C)], idx_v)
    _gather_rows(table, idx_v, out, rows_v, sem, wid * BPW)


def _sc_wide(table, ind_2d):
    kern = functools.partial(
        pl.kernel,
        out_type=jax.ShapeDtypeStruct((N, D), jnp.float32),
        mesh=_sc_mesh(),
        scratch_types=[
            pltpu.VMEM((BPW // C, C), jnp.int32),
            pltpu.VMEM((ICH, D), jnp.float32),
            pltpu.SemaphoreType.DMA,
        ],
    )
    return kern(_wide_body)(table, ind_2d)


def _narrow_body(p16, ind, p_out, idx_v, rowsp_v, sem):
    wid = _worker_id()
    pltpu.sync_copy(ind.at[pl.ds(wid * (BPW // C), BPW // C)], idx_v)
    _gather_rows(p16, idx_v, p_out, rowsp_v, sem, wid * BPW)


def _sc_narrow(p16, ind_2d):
    kern = functools.partial(
        pl.kernel,
        out_type=jax.ShapeDtypeStruct((N, 16), jnp.float32),
        mesh=_sc_mesh(),
        scratch_types=[
            pltpu.VMEM((BPW // C, C), jnp.int32),
            pltpu.VMEM((ICH, 16), jnp.float32),
            pltpu.SemaphoreType.DMA,
        ],
        compiler_params=pltpu.CompilerParams(use_tc_tiling_on_sc=False),
    )
    return kern(_narrow_body)(p16, ind_2d)


def kernel(vox_feats, pts_coors, Wpos, W1, W2, b1, b2, vox_coors, vox_numbs):
    del vox_numbs
    b2d = vox_coors[:, 0].reshape(R, C)
    x2d = vox_coors[:, 1].reshape(R, C)
    y2d = vox_coors[:, 2].reshape(R, C)
    z2d = vox_coors[:, 3].reshape(R, C)

    p16 = jnp.pad(pts_coors, ((0, 0), (0, 13)))
    wpos_p = jnp.pad(Wpos, ((0, 0), (0, 5), (0, 0)))
    b1r = b1.reshape(2, 1, D)
    b2r = b2.reshape(2, 1, D)

    ind1, c2c = _sort_a(b2d, x2d, y2d, z2d)
    p1 = _sc_narrow(p16, ind1)
    x1 = _sc_wide(vox_feats, ind1)
    ind2, ind12 = _sort_b(c2c, ind1)
    p2 = _sc_narrow(p16, ind2)
    y = _mlp(x1, p1, wpos_p[0], W1[0], W2[0], b1r[0], b2r[0])
    x2 = _sc_wide(y, ind12)
    return _mlp(x2, p2, wpos_p[1], W1[1], W2[1], b1r[1], b2r[1])


# speedup vs baseline: 2.3493x; 1.0229x over previous
import functools

import jax
import jax.numpy as jnp
from jax import lax
from jax.experimental import pallas as pl
from jax.experimental.pallas import tpu as pltpu
from jax.experimental.pallas import tpu_sc as plsc

R, C = 256, 128
N = R * C
LOG2N = 15
D = 128
GRP = 64
ORD = 7

NW = 32
BPW = N // NW
ICH = 128


def _xor_partner(a, d):
    if d < C:
        bit = (lax.broadcasted_iota(jnp.int32, (R, C), 1) & d) != 0
        return jnp.where(bit, jnp.roll(a, d, axis=1), jnp.roll(a, -d, axis=1))
    m = d // C
    bit = (lax.broadcasted_iota(jnp.int32, (R, C), 0) & m) != 0
    return jnp.where(bit, jnp.roll(a, m, axis=0), jnp.roll(a, -m, axis=0))


def _bitonic(key, val, aux=None):
    i = (lax.broadcasted_iota(jnp.int32, (R, C), 0) * C
         + lax.broadcasted_iota(jnp.int32, (R, C), 1))
    for k in range(1, LOG2N + 1):
        asc = (i & (1 << k)) == 0
        for j in range(k - 1, -1, -1):
            d = 1 << j
            kp = _xor_partner(key, d)
            vp = _xor_partner(val, d)
            bit = (i & d) != 0
            g = (key > kp) | ((key == kp) & (val > vp))
            keep = (asc ^ bit) ^ g
            key = jnp.where(keep, key, kp)
            val = jnp.where(keep, val, vp)
            if aux is not None:
                aux = jnp.where(keep, aux, _xor_partner(aux, d))
    return key, val, aux


def _morton(b, x, y, z):
    code = jnp.zeros_like(x)
    for i in range(ORD):
        code = (code
                | (((x >> i) & 1) << (3 * i))
                | (((y >> i) & 1) << (3 * i + 1))
                | (((z >> i) & 1) << (3 * i + 2)))
    return code | (b << (3 * ORD))


def _sort_a_body(b_ref, x_ref, y_ref, z_ref, ind1_ref, c2c_ref):
    b = b_ref[...]
    x = x_ref[...]
    y = y_ref[...]
    z = z_ref[...]
    code1 = _morton(b, x, y, z)
    code2 = _morton(b, x, y + 1, z + 1)
    iota = (lax.broadcasted_iota(jnp.int32, (R, C), 0) * C
            + lax.broadcasted_iota(jnp.int32, (R, C), 1))
    _, ind1, c2c = _bitonic(code1, iota, code2)
    ind1_ref[...] = ind1
    c2c_ref[...] = c2c


def _sort_b_body(c2c_ref, ind1_ref, ind2_ref, ind12_ref):
    iota = (lax.broadcasted_iota(jnp.int32, (R, C), 0) * C
            + lax.broadcasted_iota(jnp.int32, (R, C), 1))
    packed = (ind1_ref[...] << 15) | iota
    _, sv, _ = _bitonic(c2c_ref[...], packed)
    ind2_ref[...] = sv >> 15
    ind12_ref[...] = sv & (N - 1)


def _sort_a(b2, x2, y2, z2, interpret=False):
    return pl.pallas_call(
        _sort_a_body,
        out_shape=(jax.ShapeDtypeStruct((R, C), jnp.int32),
                   jax.ShapeDtypeStruct((R, C), jnp.int32)),
        interpret=interpret,
    )(b2, x2, y2, z2)


def _sort_b(c2c, ind1, interpret=False):
    return pl.pallas_call(
        _sort_b_body,
        out_shape=(jax.ShapeDtypeStruct((R, C), jnp.int32),
                   jax.ShapeDtypeStruct((R, C), jnp.int32)),
        interpret=interpret,
    )(c2c, ind1)


RB = 2048


def _mlp_body(x_ref, p_ref, wpos_ref, w1_ref, w2_ref, b1_ref, b2_ref, o_ref):
    x = x_ref[...]
    p = p_ref[...]
    pg = p.reshape(RB // GRP, GRP, 16)
    cp = (pg - jnp.mean(pg, axis=1, keepdims=True)).reshape(RB, 16)
    e = (cp[:, 0:1] * wpos_ref[0:1, :]
         + cp[:, 1:2] * wpos_ref[1:2, :]
         + cp[:, 2:3] * wpos_ref[2:3, :])
    h = x * e
    h = jnp.dot(h, w1_ref[...], preferred_element_type=jnp.float32) + b1_ref[...]
    h = jax.nn.gelu(h)
    hg = h.reshape(RB // GRP, GRP, D)
    h = (hg - jnp.mean(hg, axis=1, keepdims=True)).reshape(RB, D)
    h = jnp.dot(h, w2_ref[...], preferred_element_type=jnp.float32) + b2_ref[...]
    o_ref[...] = x + h


def _mlp(x, p16, wpos, w1, w2, b1, b2, interpret=False):
    grid = (N // RB,)
    return pl.pallas_call(
        _mlp_body,
        grid=grid,
        in_specs=[
            pl.BlockSpec((RB, D), lambda i: (i, 0)),
            pl.BlockSpec((RB, 16), lambda i: (i, 0)),
            pl.BlockSpec((8, D), lambda i: (0, 0)),
            pl.BlockSpec((D, D), lambda i: (0, 0)),
            pl.BlockSpec((D, D), lambda i: (0, 0)),
            pl.BlockSpec((1, D), lambda i: (0, 0)),
            pl.BlockSpec((1, D), lambda i: (0, 0)),
        ],
        out_specs=pl.BlockSpec((RB, D), lambda i: (i, 0)),
        out_shape=jax.ShapeDtypeStruct((N, D), jnp.float32),
        interpret=interpret,
    )(x, p16, wpos, w1, w2, b1, b2)


def _sc_mesh():
    return plsc.VectorSubcoreMesh(core_axis_name="c", subcore_axis_name="s")


def _worker_id():
    return lax.axis_index("s") * 2 + lax.axis_index("c")


NT = BPW // ICH


def _wide_body(table, ind, out, idx_v, rows_v, gsem0, gsem1, ssem0, ssem1):
    wid = _worker_id()
    base = wid * BPW
    pltpu.sync_copy(ind.at[pl.ds(wid * (BPW // C), BPW // C)], idx_v)
    gsem = (gsem0, gsem1)
    ssem = (ssem0, ssem1)
    gathers = [None, None]
    stores = [None, None]
    gathers[0] = pltpu.async_copy(table.at[idx_v.at[0]], rows_v.at[0], gsem[0])
    for t in range(NT):
        nxt = (t + 1) % 2
        if t + 1 < NT:
            if stores[nxt] is not None:
                stores[nxt].wait()
                stores[nxt] = None
            gathers[nxt] = pltpu.async_copy(
                table.at[idx_v.at[t + 1]], rows_v.at[nxt], gsem[nxt])
        gathers[t % 2].wait()
        stores[t % 2] = pltpu.async_copy(
            rows_v.at[t % 2], out.at[pl.ds(base + t * ICH, ICH)], ssem[t % 2])
    stores[(NT - 1) % 2].wait()
    if stores[NT % 2] is not None:
        stores[NT % 2].wait()


def _sc_wide(table, ind_2d):
    kern = functools.partial(
        pl.kernel,
        out_type=jax.ShapeDtypeStruct((N, D), jnp.float32),
        mesh=_sc_mesh(),
        scratch_types=[
            pltpu.VMEM((BPW // C, C), jnp.int32),
            pltpu.VMEM((2, ICH, D), jnp.float32),
            pltpu.SemaphoreType.DMA,
            pltpu.SemaphoreType.DMA,
            pltpu.SemaphoreType.DMA,
            pltpu.SemaphoreType.DMA,
        ],
    )
    return kern(_wide_body)(table, ind_2d)


def _narrow_body(p16, ind, p_out, idx_v, rows_v, sem):
    wid = _worker_id()
    pltpu.sync_copy(ind.at[pl.ds(wid * (BPW // C), BPW // C)], idx_v)
    copies = [
        pltpu.async_copy(p16.at[idx_v.at[t]], rows_v.at[pl.ds(t * ICH, ICH)], sem)
        for t in range(NT)
    ]
    for cp in copies:
        cp.wait()
    pltpu.sync_copy(rows_v, p_out.at[pl.ds(wid * BPW, BPW)])


def _sc_narrow(p16, ind_2d):
    kern = functools.partial(
        pl.kernel,
        out_type=jax.ShapeDtypeStruct((N, 16), jnp.float32),
        mesh=_sc_mesh(),
        scratch_types=[
            pltpu.VMEM((BPW // C, C), jnp.int32),
            pltpu.VMEM((BPW, 16), jnp.float32),
            pltpu.SemaphoreType.DMA,
        ],
        compiler_params=pltpu.CompilerParams(use_tc_tiling_on_sc=False),
    )
    return kern(_narrow_body)(p16, ind_2d)


def kernel(vox_feats, pts_coors, Wpos, W1, W2, b1, b2, vox_coors, vox_numbs):
    del vox_numbs
    b2d = vox_coors[:, 0].reshape(R, C)
    x2d = vox_coors[:, 1].reshape(R, C)
    y2d = vox_coors[:, 2].reshape(R, C)
    z2d = vox_coors[:, 3].reshape(R, C)

    p16 = jnp.pad(pts_coors, ((0, 0), (0, 13)))
    wpos_p = jnp.pad(Wpos, ((0, 0), (0, 5), (0, 0)))
    b1r = b1.reshape(2, 1, D)
    b2r = b2.reshape(2, 1, D)

    ind1, c2c = _sort_a(b2d, x2d, y2d, z2d)
    p1 = _sc_narrow(p16, ind1)
    x1 = _sc_wide(vox_feats, ind1)
    ind2, ind12 = _sort_b(c2c, ind1)
    p2 = _sc_narrow(p16, ind2)
    y = _mlp(x1, p1, wpos_p[0], W1[0], W2[0], b1r[0], b2r[0])
    x2 = _sc_wide(y, ind12)
    return _mlp(x2, p2, wpos_p[1], W1[1], W2[1], b1r[1], b2r[1])


# speedup vs baseline: 2.3594x; 1.0043x over previous
import functools

import jax
import jax.numpy as jnp
from jax import lax
from jax.experimental import pallas as pl
from jax.experimental.pallas import tpu as pltpu
from jax.experimental.pallas import tpu_sc as plsc

R, C = 256, 128
N = R * C
LOG2N = 15
D = 128
GRP = 64
ORD = 7

NW = 32
BPW = N // NW
ICH = 128


def _xor_partner(a, d):
    if d < C:
        bit = (lax.broadcasted_iota(jnp.int32, (R, C), 1) & d) != 0
        return jnp.where(bit, jnp.roll(a, d, axis=1), jnp.roll(a, -d, axis=1))
    m = d // C
    bit = (lax.broadcasted_iota(jnp.int32, (R, C), 0) & m) != 0
    return jnp.where(bit, jnp.roll(a, m, axis=0), jnp.roll(a, -m, axis=0))


def _bitonic(key, val, aux=None):
    i = (lax.broadcasted_iota(jnp.int32, (R, C), 0) * C
         + lax.broadcasted_iota(jnp.int32, (R, C), 1))
    for k in range(1, LOG2N + 1):
        asc = (i & (1 << k)) == 0
        for j in range(k - 1, -1, -1):
            d = 1 << j
            kp = _xor_partner(key, d)
            vp = _xor_partner(val, d)
            bit = (i & d) != 0
            g = (key > kp) | ((key == kp) & (val > vp))
            keep = (asc ^ bit) ^ g
            key = jnp.where(keep, key, kp)
            val = jnp.where(keep, val, vp)
            if aux is not None:
                aux = jnp.where(keep, aux, _xor_partner(aux, d))
    return key, val, aux


def _morton(b, x, y, z):
    code = jnp.zeros_like(x)
    for i in range(ORD):
        code = (code
                | (((x >> i) & 1) << (3 * i))
                | (((y >> i) & 1) << (3 * i + 1))
                | (((z >> i) & 1) << (3 * i + 2)))
    return code | (b << (3 * ORD))


def _sort_a_body(b_ref, x_ref, y_ref, z_ref, ind1_ref, c2c_ref):
    b = b_ref[...]
    x = x_ref[...]
    y = y_ref[...]
    z = z_ref[...]
    code1 = _morton(b, x, y, z)
    code2 = _morton(b, x, y + 1, z + 1)
    iota = (lax.broadcasted_iota(jnp.int32, (R, C), 0) * C
            + lax.broadcasted_iota(jnp.int32, (R, C), 1))
    _, ind1, c2c = _bitonic(code1, iota, code2)
    ind1_ref[...] = ind1
    c2c_ref[...] = c2c


def _sort_b_body(c2c_ref, ind1_ref, ind2_ref, ind12_ref):
    iota = (lax.broadcasted_iota(jnp.int32, (R, C), 0) * C
            + lax.broadcasted_iota(jnp.int32, (R, C), 1))
    packed = (ind1_ref[...] << 15) | iota
    _, sv, _ = _bitonic(c2c_ref[...], packed)
    ind2_ref[...] = sv >> 15
    ind12_ref[...] = sv & (N - 1)


def _sort_a(b2, x2, y2, z2, interpret=False):
    return pl.pallas_call(
        _sort_a_body,
        out_shape=(jax.ShapeDtypeStruct((R, C), jnp.int32),
                   jax.ShapeDtypeStruct((R, C), jnp.int32)),
        interpret=interpret,
    )(b2, x2, y2, z2)


def _sort_b(c2c, ind1, interpret=False):
    return pl.pallas_call(
        _sort_b_body,
        out_shape=(jax.ShapeDtypeStruct((R, C), jnp.int32),
                   jax.ShapeDtypeStruct((R, C), jnp.int32)),
        interpret=interpret,
    )(c2c, ind1)


RB = 2048


def _proj_body(p8_ref, wpos0_ref, wpos1_ref, q0_ref, q1_ref):
    p8 = p8_ref[...]
    q0_ref[...] = jnp.dot(p8, wpos0_ref[...], preferred_element_type=jnp.float32)
    q1_ref[...] = jnp.dot(p8, wpos1_ref[...], preferred_element_type=jnp.float32)


def _proj(p8, wpos0, wpos1, interpret=False):
    return pl.pallas_call(
        _proj_body,
        grid=(N // RB,),
        in_specs=[
            pl.BlockSpec((RB, 8), lambda i: (i, 0)),
            pl.BlockSpec((8, D), lambda i: (0, 0)),
            pl.BlockSpec((8, D), lambda i: (0, 0)),
        ],
        out_specs=(pl.BlockSpec((RB, D), lambda i: (i, 0)),
                   pl.BlockSpec((RB, D), lambda i: (i, 0))),
        out_shape=(jax.ShapeDtypeStruct((N, D), jnp.float32),
                   jax.ShapeDtypeStruct((N, D), jnp.float32)),
        interpret=interpret,
    )(p8, wpos0, wpos1)


def _mlp_body(x_ref, q_ref, w1_ref, w2_ref, b1_ref, b2_ref, o_ref):
    x = x_ref[...]
    qg = q_ref[...].reshape(RB // GRP, GRP, D)
    e = (qg - jnp.mean(qg, axis=1, keepdims=True)).reshape(RB, D)
    h = x * e
    h = jnp.dot(h, w1_ref[...], preferred_element_type=jnp.float32) + b1_ref[...]
    h = jax.nn.gelu(h)
    hg = h.reshape(RB // GRP, GRP, D)
    h = (hg - jnp.mean(hg, axis=1, keepdims=True)).reshape(RB, D)
    h = jnp.dot(h, w2_ref[...], preferred_element_type=jnp.float32) + b2_ref[...]
    o_ref[...] = x + h


def _mlp(x, q, w1, w2, b1, b2, interpret=False):
    grid = (N // RB,)
    return pl.pallas_call(
        _mlp_body,
        grid=grid,
        in_specs=[
            pl.BlockSpec((RB, D), lambda i: (i, 0)),
            pl.BlockSpec((RB, D), lambda i: (i, 0)),
            pl.BlockSpec((D, D), lambda i: (0, 0)),
            pl.BlockSpec((D, D), lambda i: (0, 0)),
            pl.BlockSpec((1, D), lambda i: (0, 0)),
            pl.BlockSpec((1, D), lambda i: (0, 0)),
        ],
        out_specs=pl.BlockSpec((RB, D), lambda i: (i, 0)),
        out_shape=jax.ShapeDtypeStruct((N, D), jnp.float32),
        interpret=interpret,
    )(x, q, w1, w2, b1, b2)


def _sc_mesh():
    return plsc.VectorSubcoreMesh(core_axis_name="c", subcore_axis_name="s")


def _worker_id():
    return lax.axis_index("s") * 2 + lax.axis_index("c")


NT = BPW // ICH


def _wide_body(table, ind, out, idx_v, rows_v, gsem0, gsem1, ssem0, ssem1):
    wid = _worker_id()
    base = wid * BPW
    pltpu.sync_copy(ind.at[pl.ds(wid * (BPW // C), BPW // C)], idx_v)
    gsem = (gsem0, gsem1)
    ssem = (ssem0, ssem1)
    gathers = [None, None]
    stores = [None, None]
    gathers[0] = pltpu.async_copy(table.at[idx_v.at[0]], rows_v.at[0], gsem[0])
    for t in range(NT):
        nxt = (t + 1) % 2
        if t + 1 < NT:
            if stores[nxt] is not None:
                stores[nxt].wait()
                stores[nxt] = None
            gathers[nxt] = pltpu.async_copy(
                table.at[idx_v.at[t + 1]], rows_v.at[nxt], gsem[nxt])
        gathers[t % 2].wait()
        stores[t % 2] = pltpu.async_copy(
            rows_v.at[t % 2], out.at[pl.ds(base + t * ICH, ICH)], ssem[t % 2])
    stores[(NT - 1) % 2].wait()
    if stores[NT % 2] is not None:
        stores[NT % 2].wait()


def _sc_wide(table, ind_2d):
    kern = functools.partial(
        pl.kernel,
        out_type=jax.ShapeDtypeStruct((N, D), jnp.float32),
        mesh=_sc_mesh(),
        scratch_types=[
            pltpu.VMEM((BPW // C, C), jnp.int32),
            pltpu.VMEM((2, ICH, D), jnp.float32),
            pltpu.SemaphoreType.DMA,
            pltpu.SemaphoreType.DMA,
            pltpu.SemaphoreType.DMA,
            pltpu.SemaphoreType.DMA,
        ],
    )
    return kern(_wide_body)(table, ind_2d)


def kernel(vox_feats, pts_coors, Wpos, W1, W2, b1, b2, vox_coors, vox_numbs):
    del vox_numbs
    b2d = vox_coors[:, 0].reshape(R, C)
    x2d = vox_coors[:, 1].reshape(R, C)
    y2d = vox_coors[:, 2].reshape(R, C)
    z2d = vox_coors[:, 3].reshape(R, C)

    p8 = jnp.pad(pts_coors, ((0, 0), (0, 5)))
    wpos_p = jnp.pad(Wpos, ((0, 0), (0, 5), (0, 0)))
    b1r = b1.reshape(2, 1, D)
    b2r = b2.reshape(2, 1, D)

    q0, q1 = _proj(p8, wpos_p[0], wpos_p[1])
    ind1, c2c = _sort_a(b2d, x2d, y2d, z2d)
    e1 = _sc_wide(q0, ind1)
    x1 = _sc_wide(vox_feats, ind1)
    ind2, ind12 = _sort_b(c2c, ind1)
    e2 = _sc_wide(q1, ind2)
    y = _mlp(x1, e1, W1[0], W2[0], b1r[0], b2r[0])
    x2 = _sc_wide(y, ind12)
    return _mlp(x2, e2, W1[1], W2[1], b1r[1], b2r[1])


# speedup vs baseline: 2.7005x; 1.1446x over previous
import functools

import jax
import jax.numpy as jnp
from jax import lax
from jax.experimental import pallas as pl
from jax.experimental.pallas import tpu as pltpu
from jax.experimental.pallas import tpu_sc as plsc

R, C = 256, 128
N = R * C
LOG2N = 15
D = 128
GRP = 64
ORD = 7

NW = 32
BPW = N // NW
ICH = 128


def _xor_partner(a, d):
    if d < C:
        bit = (lax.broadcasted_iota(jnp.int32, (R, C), 1) & d) != 0
        return jnp.where(bit, jnp.roll(a, d, axis=1), jnp.roll(a, -d, axis=1))
    m = d // C
    bit = (lax.broadcasted_iota(jnp.int32, (R, C), 0) & m) != 0
    return jnp.where(bit, jnp.roll(a, m, axis=0), jnp.roll(a, -m, axis=0))


def _bitonic(key, val, aux=None):
    i = (lax.broadcasted_iota(jnp.int32, (R, C), 0) * C
         + lax.broadcasted_iota(jnp.int32, (R, C), 1))
    for k in range(1, LOG2N + 1):
        asc = (i & (1 << k)) == 0
        for j in range(k - 1, -1, -1):
            d = 1 << j
            kp = _xor_partner(key, d)
            vp = _xor_partner(val, d)
            bit = (i & d) != 0
            g = (key > kp) | ((key == kp) & (val > vp))
            keep = (asc ^ bit) ^ g
            key = jnp.where(keep, key, kp)
            val = jnp.where(keep, val, vp)
            if aux is not None:
                aux = jnp.where(keep, aux, _xor_partner(aux, d))
    return key, val, aux


def _morton(b, x, y, z):
    code = jnp.zeros_like(x)
    for i in range(ORD):
        code = (code
                | (((x >> i) & 1) << (3 * i))
                | (((y >> i) & 1) << (3 * i + 1))
                | (((z >> i) & 1) << (3 * i + 2)))
    return code | (b << (3 * ORD))


def _sort_a_body(coors_ref, ind1_ref, c2c_ref):
    b = coors_ref[0 * R:1 * R, :]
    x = coors_ref[1 * R:2 * R, :]
    y = coors_ref[2 * R:3 * R, :]
    z = coors_ref[3 * R:4 * R, :]
    code1 = _morton(b, x, y, z)
    code2 = _morton(b, x, y + 1, z + 1)
    iota = (lax.broadcasted_iota(jnp.int32, (R, C), 0) * C
            + lax.broadcasted_iota(jnp.int32, (R, C), 1))
    _, ind1, c2c = _bitonic(code1, iota, code2)
    ind1_ref[...] = ind1
    c2c_ref[...] = c2c


def _sort_b_body(c2c_ref, ind1_ref, ind2_ref, ind12_ref):
    iota = (lax.broadcasted_iota(jnp.int32, (R, C), 0) * C
            + lax.broadcasted_iota(jnp.int32, (R, C), 1))
    packed = (ind1_ref[...] << 15) | iota
    _, sv, _ = _bitonic(c2c_ref[...], packed)
    ind2_ref[...] = sv >> 15
    ind12_ref[...] = sv & (N - 1)


def _sort_a(coors_t, interpret=False):
    return pl.pallas_call(
        _sort_a_body,
        out_shape=(jax.ShapeDtypeStruct((R, C), jnp.int32),
                   jax.ShapeDtypeStruct((R, C), jnp.int32)),
        interpret=interpret,
    )(coors_t)


def _sort_b(c2c, ind1, interpret=False):
    return pl.pallas_call(
        _sort_b_body,
        out_shape=(jax.ShapeDtypeStruct((R, C), jnp.int32),
                   jax.ShapeDtypeStruct((R, C), jnp.int32)),
        interpret=interpret,
    )(c2c, ind1)


RB = 2048


def _proj_body(pt_ref, wpos0_ref, wpos1_ref, q0_ref, q1_ref):
    pt = pt_ref[...]
    dn = (((0,), (0,)), ((), ()))
    q0_ref[...] = lax.dot_general(pt, wpos0_ref[...], dn,
                                  preferred_element_type=jnp.float32)
    q1_ref[...] = lax.dot_general(pt, wpos1_ref[...], dn,
                                  preferred_element_type=jnp.float32)


def _proj(pts_t8, wpos0, wpos1, interpret=False):
    return pl.pallas_call(
        _proj_body,
        grid=(N // RB,),
        in_specs=[
            pl.BlockSpec((8, RB), lambda i: (0, i)),
            pl.BlockSpec((8, D), lambda i: (0, 0)),
            pl.BlockSpec((8, D), lambda i: (0, 0)),
        ],
        out_specs=(pl.BlockSpec((RB, D), lambda i: (i, 0)),
                   pl.BlockSpec((RB, D), lambda i: (i, 0))),
        out_shape=(jax.ShapeDtypeStruct((N, D), jnp.float32),
                   jax.ShapeDtypeStruct((N, D), jnp.float32)),
        interpret=interpret,
    )(pts_t8, wpos0, wpos1)


def _mlp_body(x_ref, q_ref, w1_ref, w2_ref, b1_ref, b2_ref, o_ref):
    x = x_ref[...]
    qg = q_ref[...].reshape(RB // GRP, GRP, D)
    e = (qg - jnp.mean(qg, axis=1, keepdims=True)).reshape(RB, D)
    h = x * e
    h = jnp.dot(h, w1_ref[...], preferred_element_type=jnp.float32) + b1_ref[...]
    h = jax.nn.gelu(h)
    hg = h.reshape(RB // GRP, GRP, D)
    h = (hg - jnp.mean(hg, axis=1, keepdims=True)).reshape(RB, D)
    h = jnp.dot(h, w2_ref[...], preferred_element_type=jnp.float32) + b2_ref[...]
    o_ref[...] = x + h


def _mlp(x, q, w1, w2, b1, b2, interpret=False):
    grid = (N // RB,)
    return pl.pallas_call(
        _mlp_body,
        grid=grid,
        in_specs=[
            pl.BlockSpec((RB, D), lambda i: (i, 0)),
            pl.BlockSpec((RB, D), lambda i: (i, 0)),
            pl.BlockSpec((D, D), lambda i: (0, 0)),
            pl.BlockSpec((D, D), lambda i: (0, 0)),
            pl.BlockSpec((1, D), lambda i: (0, 0)),
            pl.BlockSpec((1, D), lambda i: (0, 0)),
        ],
        out_specs=pl.BlockSpec((RB, D), lambda i: (i, 0)),
        out_shape=jax.ShapeDtypeStruct((N, D), jnp.float32),
        interpret=interpret,
    )(x, q, w1, w2, b1, b2)


def _sc_mesh():
    return plsc.VectorSubcoreMesh(core_axis_name="c", subcore_axis_name="s")


def _worker_id():
    return lax.axis_index("s") * 2 + lax.axis_index("c")


NT = BPW // ICH


def _wide_body(table, ind, out, idx_v, rows_v, gsem0, gsem1, ssem0, ssem1):
    wid = _worker_id()
    base = wid * BPW
    pltpu.sync_copy(ind.at[pl.ds(wid * (BPW // C), BPW // C)], idx_v)
    gsem = (gsem0, gsem1)
    ssem = (ssem0, ssem1)
    gathers = [None, None]
    stores = [None, None]
    gathers[0] = pltpu.async_copy(table.at[idx_v.at[0]], rows_v.at[0], gsem[0])
    for t in range(NT):
        nxt = (t + 1) % 2
        if t + 1 < NT:
            if stores[nxt] is not None:
                stores[nxt].wait()
                stores[nxt] = None
            gathers[nxt] = pltpu.async_copy(
                table.at[idx_v.at[t + 1]], rows_v.at[nxt], gsem[nxt])
        gathers[t % 2].wait()
        stores[t % 2] = pltpu.async_copy(
            rows_v.at[t % 2], out.at[pl.ds(base + t * ICH, ICH)], ssem[t % 2])
    stores[(NT - 1) % 2].wait()
    if stores[NT % 2] is not None:
        stores[NT % 2].wait()


def _sc_wide(table, ind_2d):
    kern = functools.partial(
        pl.kernel,
        out_type=jax.ShapeDtypeStruct((N, D), jnp.float32),
        mesh=_sc_mesh(),
        scratch_types=[
            pltpu.VMEM((BPW // C, C), jnp.int32),
            pltpu.VMEM((2, ICH, D), jnp.float32),
            pltpu.SemaphoreType.DMA,
            pltpu.SemaphoreType.DMA,
            pltpu.SemaphoreType.DMA,
            pltpu.SemaphoreType.DMA,
        ],
    )
    return kern(_wide_body)(table, ind_2d)


def kernel(vox_feats, pts_coors, Wpos, W1, W2, b1, b2, vox_coors, vox_numbs):
    del vox_numbs
    coors_t = vox_coors.T.reshape(4 * R, C)
    pts_t8 = jnp.pad(pts_coors.T, ((0, 5), (0, 0)))

    wpos_p = jnp.pad(Wpos, ((0, 0), (0, 5), (0, 0)))
    b1r = b1.reshape(2, 1, D)
    b2r = b2.reshape(2, 1, D)

    q0, q1 = _proj(pts_t8, wpos_p[0], wpos_p[1])
    ind1, c2c = _sort_a(coors_t)
    e1 = _sc_wide(q0, ind1)
    x1 = _sc_wide(vox_feats, ind1)
    ind2, ind12 = _sort_b(c2c, ind1)
    e2 = _sc_wide(q1, ind2)
    y = _mlp(x1, e1, W1[0], W2[0], b1r[0], b2r[0])
    x2 = _sc_wide(y, ind12)
    return _mlp(x2, e2, W1[1], W2[1], b1r[1], b2r[1])
